# Initial kernel scaffold; baseline (speedup 1.0000x reference)
import jax
import jax.numpy as jnp
from jax.experimental import pallas as pl


def kernel(max_values, argmax):
    raise NotImplementedError("write your pallas kernel here")


# speedup vs baseline: 19.8394x; 19.8394x over previous
import jax
import jax.numpy as jnp
from jax import lax
from jax.experimental import pallas as pl
from jax.experimental.pallas import tpu as pltpu
from jax.experimental.pallas import tpu_sc as plsc

B, H, W, C = 8, 112, 112, 96
IMG_IN = H * W * C
IMG_OUT = 4 * IMG_IN
TOTAL_OUT = B * IMG_OUT

NC, NS = 2, 16
NREG = 3
REGION = IMG_OUT // NREG
TASKS = (B // NC) * NREG

CHUNK = 5376
PAIRS_PER_TILE = IMG_IN // NS
NCHUNK = PAIRS_PER_TILE // CHUNK
CH_PER_BATCH = IMG_IN // CHUNK

TRASH_PER_TILE = 1024
TRASH = NS * TRASH_PER_TILE
SPMEM_WORDS = REGION + TRASH

OUT_PER_TILE = REGION // NS
CPBUF = 12544
NCP = OUT_PER_TILE // CPBUF


def _body(val_hbm, idx_hbm, out_hbm, shared, idxb, valb, sidxb, cbuf):
    c = lax.axis_index("c")
    t = lax.axis_index("s")
    lane = lax.iota(jnp.int32, 16)

    def task_body(r, carry):
        b = c * (B // NC) + r // NREG
        q = r % NREG
        lo = q * REGION
        hi = lo + REGION

        def zfill(i, carry2):
            cbuf[pl.ds(i * 16, 16)] = jnp.zeros((16,), jnp.float32)
            return carry2

        lax.fori_loop(0, CPBUF // 16, zfill, 0)
        for k in range(NCP):
            pltpu.sync_copy(
                cbuf, shared.at[pl.ds(t * OUT_PER_TILE + k * CPBUF, CPBUF)])
        plsc.subcore_barrier()

        def chunk_body(k, carry2):
            off = (b * CH_PER_BATCH + k * NS + t) * CHUNK
            pltpu.sync_copy(idx_hbm.at[pl.ds(off, CHUNK)], idxb)
            pltpu.sync_copy(val_hbm.at[pl.ds(off, CHUNK)], valb)

            def vec_body(j, carry3):
                iv = idxb[pl.ds(j * 16, 16)]
                m = (iv >= lo) & (iv < hi)
                tr = (REGION + t * TRASH_PER_TILE
                      + ((j * 16) & (TRASH_PER_TILE - 1)) + lane)
                sidxb[pl.ds(j * 16, 16)] = jnp.where(m, iv - lo, tr)
                return carry3

            lax.fori_loop(0, CHUNK // 16, vec_body, 0)
            pltpu.sync_copy(valb, shared.at[sidxb], add=True)
            return carry2

        lax.fori_loop(0, NCHUNK, chunk_body, 0)
        plsc.subcore_barrier()

        g0 = b * IMG_OUT + lo + t * OUT_PER_TILE
        def cp_body(k, carry2):
            pltpu.sync_copy(
                shared.at[pl.ds(t * OUT_PER_TILE + k * CPBUF, CPBUF)], cbuf)
            pltpu.sync_copy(cbuf, out_hbm.at[pl.ds(g0 + k * CPBUF, CPBUF)])
            return carry2

        lax.fori_loop(0, NCP, cp_body, 0)
        return carry

    lax.fori_loop(0, TASKS, task_body, 0)


def kernel(max_values, argmax):
    vals = max_values.reshape(B * IMG_IN)
    idx = argmax.astype(jnp.int32).reshape(B * IMG_IN)
    run = pl.kernel(
        _body,
        out_type=jax.ShapeDtypeStruct((TOTAL_OUT,), jnp.float32),
        mesh=plsc.VectorSubcoreMesh(
            core_axis_name="c", subcore_axis_name="s",
            num_cores=NC, num_subcores=NS),
        scratch_types=[
            pltpu.MemorySpace.VMEM_SHARED((SPMEM_WORDS,), jnp.float32),
            pltpu.MemorySpace.VMEM((CHUNK,), jnp.int32),
            pltpu.MemorySpace.VMEM((CHUNK,), jnp.float32),
            pltpu.MemorySpace.VMEM((CHUNK,), jnp.int32),
            pltpu.MemorySpace.VMEM((CPBUF,), jnp.float32),
        ],
    )
    out = run(vals, idx)
    return out.reshape(B, 2 * H, 2 * W, C)


# speedup vs baseline: 26.4175x; 1.3316x over previous
import jax
import jax.numpy as jnp
from jax import lax
from jax.experimental import pallas as pl
from jax.experimental.pallas import tpu as pltpu
from jax.experimental.pallas import tpu_sc as plsc

B, H, W, C = 8, 112, 112, 96
IMG_IN = H * W * C
IMG_OUT = 4 * IMG_IN
TOTAL_OUT = B * IMG_OUT

NC, NS = 2, 16
NREG = 3
REGION = IMG_OUT // NREG
TASKS = (B // NC) * NREG

CHUNK = 5376
PAIRS_PER_TILE = IMG_IN // NS
NCHUNK = PAIRS_PER_TILE // CHUNK
CH_PER_BATCH = IMG_IN // CHUNK

TRASH_PER_TILE = 1024
TRASH = NS * TRASH_PER_TILE
SPMEM_WORDS = REGION + TRASH

OUT_PER_TILE = REGION // NS
CPBUF = 6272
NCP = OUT_PER_TILE // CPBUF


def _body(val_hbm, idx_hbm, out_hbm, shared,
          idxb0, idxb1, valb0, valb1, cbuf, sl0, sl1, ss0, ss1):
    c = lax.axis_index("c")
    t = lax.axis_index("s")
    lane = lax.iota(jnp.int32, 16)
    idxb = (idxb0, idxb1)
    valb = (valb0, valb1)
    sl = (sl0, sl1)
    ss = (ss0, ss1)

    def zfill(i, carry):
        cbuf[pl.ds(i * 16, 16)] = jnp.zeros((16,), jnp.float32)
        return carry

    lax.fori_loop(0, CPBUF // 16, zfill, 0)

    def task_body(r, carry):
        b = c * (B // NC) + r // NREG
        q = r % NREG
        lo = q * REGION
        hi = lo + REGION

        for k in range(NCP):
            pltpu.sync_copy(
                cbuf, shared.at[pl.ds(t * OUT_PER_TILE + k * CPBUF, CPBUF)])
        plsc.subcore_barrier()

        def start_load(k):
            p = k % 2
            off = (b * CH_PER_BATCH + k * NS + t) * CHUNK
            hi_ = pltpu.async_copy(idx_hbm.at[pl.ds(off, CHUNK)], idxb[p],
                                   sl[p])
            hv_ = pltpu.async_copy(val_hbm.at[pl.ds(off, CHUNK)], valb[p],
                                   sl[p])
            return hi_, hv_

        h_load = start_load(0)
        h_scat = None
        for k in range(NCHUNK):
            p = k % 2
            h_load[0].wait()
            h_load[1].wait()

            def vec_body(j, carry3, _ib=idxb[p]):
                iv = _ib[pl.ds(j * 16, 16)]
                m = (iv >= lo) & (iv < hi)
                tr = (REGION + t * TRASH_PER_TILE
                      + ((j * 16) & (TRASH_PER_TILE - 1)) + lane)
                _ib[pl.ds(j * 16, 16)] = jnp.where(m, iv - lo, tr)
                return carry3

            lax.fori_loop(0, CHUNK // 16, vec_body, 0)
            if h_scat is not None:
                h_scat.wait()
            h_scat = pltpu.async_copy(valb[p], shared.at[idxb[p]], ss[p],
                                      add=True)
            if k + 1 < NCHUNK:
                h_load = start_load(k + 1)
        h_scat.wait()
        plsc.subcore_barrier()

        g0 = b * IMG_OUT + lo + t * OUT_PER_TILE
        pltpu.sync_copy(shared.at[pl.ds(t * OUT_PER_TILE, OUT_PER_TILE)],
                        out_hbm.at[pl.ds(g0, OUT_PER_TILE)])
        return carry

    lax.fori_loop(0, TASKS, task_body, 0)


def kernel(max_values, argmax):
    vals = max_values.reshape(B * IMG_IN)
    idx = argmax.astype(jnp.int32).reshape(B * IMG_IN)
    run = pl.kernel(
        _body,
        out_type=jax.ShapeDtypeStruct((TOTAL_OUT,), jnp.float32),
        mesh=plsc.VectorSubcoreMesh(
            core_axis_name="c", subcore_axis_name="s",
            num_cores=NC, num_subcores=NS),
        scratch_types=[
            pltpu.MemorySpace.VMEM_SHARED((SPMEM_WORDS,), jnp.float32),
            pltpu.MemorySpace.VMEM((CHUNK,), jnp.int32),
            pltpu.MemorySpace.VMEM((CHUNK,), jnp.int32),
            pltpu.MemorySpace.VMEM((CHUNK,), jnp.float32),
            pltpu.MemorySpace.VMEM((CHUNK,), jnp.float32),
            pltpu.MemorySpace.VMEM((CPBUF,), jnp.float32),
            pltpu.SemaphoreType.DMA,
            pltpu.SemaphoreType.DMA,
            pltpu.SemaphoreType.DMA,
            pltpu.SemaphoreType.DMA,
        ],
    )
    out = run(vals, idx)
    return out.reshape(B, 2 * H, 2 * W, C)
